# baseline (speedup 1.0000x reference)
import functools

import jax
import jax.numpy as jnp
from jax import lax
from jax.experimental import pallas as pl
from jax.experimental.pallas import tpu as pltpu
from jax.experimental.pallas import tpu_sc as plsc

DIM = 64
BATCH = 4096
HIST = 50
NC, NS = 2, 16
NW = NC * NS
RPW = BATCH // NW
NB = 2
NPAIR = HIST // NB

_mesh = plsc.VectorSubcoreMesh(core_axis_name="c", subcore_axis_name="s")


@functools.partial(
    pl.kernel,
    mesh=_mesh,
    out_type=jax.ShapeDtypeStruct((HIST, DIM, BATCH), jnp.float32),
    scratch_types=[
        pltpu.VMEM((HIST, RPW), jnp.int32),
        pltpu.VMEM((NB, RPW, DIM), jnp.float32),
        pltpu.VMEM((NB, DIM, RPW), jnp.float32),
        [pltpu.SemaphoreType.DMA] * NB,
        [pltpu.SemaphoreType.DMA] * NB,
    ],
    compiler_params=pltpu.CompilerParams(
        use_tc_tiling_on_sc=False, needs_layout_passes=False),
)
def _emb_gather(inpt_hbm, table_hbm, out_hbm, idx_v, gbuf, tbuf, gsems, osems):
    wid = lax.axis_index("s") * NC + lax.axis_index("c")
    b0 = wid * RPW
    pltpu.sync_copy(inpt_hbm.at[:, pl.ds(b0, RPW)], idx_v)

    def gather_start(h, b):
        pltpu.async_copy(table_hbm.at[idx_v.at[h]], gbuf.at[b], gsems[b])

    def gather_wait(h, b):
        pltpu.make_async_copy(
            table_hbm.at[idx_v.at[h]], gbuf.at[b], gsems[b]).wait()

    def out_start(h, b):
        pltpu.async_copy(
            tbuf.at[b], out_hbm.at[h, :, pl.ds(b0, RPW)], osems[b])

    def out_wait(h, b):
        pltpu.make_async_copy(
            tbuf.at[b], out_hbm.at[h, :, pl.ds(b0, RPW)], osems[b]).wait()

    iota = lax.iota(jnp.int32, 16)
    d_vecs = [iota + d0 for d0 in range(0, DIM, 16)]

    def transpose_slab(b):
        def rbody(r, carry):
            r_splat = jnp.full((16,), 0, jnp.int32) + r
            for k in range(DIM // 16):
                v = gbuf[b, r, pl.ds(k * 16, 16)]
                plsc.store_scatter(tbuf.at[b], [d_vecs[k], r_splat], v)
            return carry

        lax.fori_loop(0, RPW, rbody, 0)

    for b in range(NB):
        gather_start(b, b)

    def pair(g, carry):
        for b in range(NB):
            h = g * NB + b
            gather_wait(h, b)

            @pl.when(h >= NB)
            def _():
                out_wait(h - NB, b)

            transpose_slab(b)
            out_start(h, b)

            @pl.when(h + NB < HIST)
            def _():
                gather_start(h + NB, b)

        return carry

    lax.fori_loop(0, NPAIR, pair, 0)

    for h in range(HIST - NB, HIST):
        out_wait(h, h % NB)


def kernel(inp, table):
    out_t = _emb_gather(inp.T, table)
    return jnp.transpose(out_t, (2, 0, 1))


# speedup vs baseline: 1.6371x; 1.6371x over previous
import functools

import jax
import jax.numpy as jnp
from jax import lax
from jax.experimental import pallas as pl
from jax.experimental.pallas import tpu as pltpu
from jax.experimental.pallas import tpu_sc as plsc

DIM = 64
BATCH = 4096
HIST = 50
NC, NS = 2, 16
NW = NC * NS
RPW = BATCH // NW
NB = 5
NG = HIST // NB

_mesh = plsc.VectorSubcoreMesh(core_axis_name="c", subcore_axis_name="s")


@functools.partial(
    pl.kernel,
    mesh=_mesh,
    out_type=jax.ShapeDtypeStruct((HIST, BATCH, DIM), jnp.float32),
    scratch_types=[
        pltpu.VMEM((HIST, RPW), jnp.int32),
        pltpu.VMEM((NB, RPW, DIM), jnp.float32),
        [pltpu.SemaphoreType.DMA] * NB,
        [pltpu.SemaphoreType.DMA] * NB,
    ],
    compiler_params=pltpu.CompilerParams(
        use_tc_tiling_on_sc=False, needs_layout_passes=False),
)
def _emb_gather(inpt_hbm, table_hbm, out_hbm, idx_v, gbuf, gsems, osems):
    wid = lax.axis_index("s") * NC + lax.axis_index("c")
    b0 = wid * RPW
    pltpu.sync_copy(inpt_hbm.at[:, pl.ds(b0, RPW)], idx_v)

    def gather_start(h, b):
        pltpu.async_copy(table_hbm.at[idx_v.at[h]], gbuf.at[b], gsems[b])

    def gather_wait(h, b):
        pltpu.make_async_copy(
            table_hbm.at[idx_v.at[h]], gbuf.at[b], gsems[b]).wait()

    def out_start(h, b):
        pltpu.async_copy(
            gbuf.at[b], out_hbm.at[h, pl.ds(b0, RPW), :], osems[b])

    def out_wait(h, b):
        pltpu.make_async_copy(
            gbuf.at[b], out_hbm.at[h, pl.ds(b0, RPW), :], osems[b]).wait()

    for b in range(NB - 1):
        gather_start(b, b)

    def group(g, carry):
        for b in range(NB):
            h = g * NB + b
            gather_wait(h, b)
            out_start(h, b)
            bp = (b - 1) % NB

            @pl.when(h >= 1)
            def _():
                out_wait(h - 1, bp)

            bn = (b + NB - 1) % NB

            @pl.when(h + NB - 1 < HIST)
            def _():
                gather_start(h + NB - 1, bn)

        return carry

    lax.fori_loop(0, NG, group, 0)

    out_wait(HIST - 1, (HIST - 1) % NB)


def kernel(inp, table):
    out_t = _emb_gather(inp.T, table)
    return jnp.transpose(out_t, (1, 0, 2))


# speedup vs baseline: 2.1993x; 1.3434x over previous
import functools

import jax
import jax.numpy as jnp
from jax import lax
from jax.experimental import pallas as pl
from jax.experimental.pallas import tpu as pltpu
from jax.experimental.pallas import tpu_sc as plsc

DIM = 64
BATCH = 4096
HIST = 50
NC, NS = 2, 16
NW = NC * NS
RPW = BATCH // NW
NBG = 3
NBT = 2
TP = 129
STEP = 6
NGRP = 48 // STEP

_mesh = plsc.VectorSubcoreMesh(core_axis_name="c", subcore_axis_name="s")


@functools.partial(
    pl.kernel,
    mesh=_mesh,
    out_type=jax.ShapeDtypeStruct((HIST, DIM, BATCH), jnp.float32),
    scratch_types=[
        pltpu.VMEM((HIST, RPW), jnp.int32),
        pltpu.VMEM((NBG, RPW, DIM), jnp.float32),
        pltpu.VMEM((NBT, DIM, TP), jnp.float32),
        [pltpu.SemaphoreType.DMA] * NBG,
        [pltpu.SemaphoreType.DMA] * NBT,
    ],
    compiler_params=pltpu.CompilerParams(
        use_tc_tiling_on_sc=False, needs_layout_passes=False),
)
def _emb_gather(inpt_hbm, table_hbm, out_hbm, idx_v, gbuf, tbuf, gsems, osems):
    wid = lax.axis_index("s") * NC + lax.axis_index("c")
    b0 = wid * RPW
    pltpu.sync_copy(inpt_hbm.at[:, pl.ds(b0, RPW)], idx_v)

    def gather_start(h, bg):
        pltpu.async_copy(table_hbm.at[idx_v.at[h]], gbuf.at[bg], gsems[bg])

    def gather_wait(h, bg):
        pltpu.make_async_copy(
            table_hbm.at[idx_v.at[h]], gbuf.at[bg], gsems[bg]).wait()

    def out_start(h, bt):
        pltpu.async_copy(tbuf.at[bt, :, pl.ds(0, RPW)],
                         out_hbm.at[h, :, pl.ds(b0, RPW)], osems[bt])

    def out_wait(h, bt):
        pltpu.make_async_copy(
            tbuf.at[bt, :, pl.ds(0, RPW)],
            out_hbm.at[h, :, pl.ds(b0, RPW)], osems[bt]).wait()

    iota = lax.iota(jnp.int32, 16)
    d_vecs = [iota + d0 for d0 in range(0, DIM, 16)]
    zeros16 = jnp.full((16,), 0, jnp.int32)

    def transpose_slab(bg, bt):
        @plsc.parallel_loop(0, RPW, unroll=8)
        def _(r):
            r_splat = zeros16 + r
            for k in range(DIM // 16):
                v = gbuf[bg, r, pl.ds(k * 16, 16)]
                plsc.store_scatter(tbuf.at[bt], [d_vecs[k], r_splat], v)

    def step(h, bg, bt):
        static = isinstance(h, int)
        gather_wait(h, bg)

        if static:
            if h >= NBT:
                out_wait(h - NBT, bt)
        else:
            @pl.when(h >= NBT)
            def _():
                out_wait(h - NBT, bt)

        transpose_slab(bg, bt)
        out_start(h, bt)

        if static:
            if h + NBG < HIST:
                gather_start(h + NBG, bg)
        else:
            @pl.when(h + NBG < HIST)
            def _():
                gather_start(h + NBG, bg)

    for bg in range(NBG):
        gather_start(bg, bg)

    def group(g, carry):
        for b in range(STEP):
            step(g * STEP + b, b % NBG, b % NBT)
        return carry

    lax.fori_loop(0, NGRP, group, 0)

    for h in (48, 49):
        step(h, h % NBG, h % NBT)

    for h in (48, 49):
        out_wait(h, h % NBT)


def kernel(inp, table):
    out_t = _emb_gather(inp.T, table)
    return jnp.transpose(out_t, (2, 0, 1))


# speedup vs baseline: 3.1784x; 1.4452x over previous
import functools

import jax
import jax.numpy as jnp
from jax import lax
from jax.experimental import pallas as pl
from jax.experimental.pallas import tpu as pltpu
from jax.experimental.pallas import tpu_sc as plsc

DIM = 64
BATCH = 4096
HIST = 50
NC, NS = 2, 16
NW = NC * NS
RPW = BATCH // NW
NBG = 3
NBT = 2
TP = 129
STEP = 6
NGRP = 48 // STEP

_mesh = plsc.VectorSubcoreMesh(core_axis_name="c", subcore_axis_name="s")


@functools.partial(
    pl.kernel,
    mesh=_mesh,
    out_type=jax.ShapeDtypeStruct((HIST, DIM // 8, NW, 8, RPW), jnp.float32),
    scratch_types=[
        pltpu.VMEM((HIST, RPW), jnp.int32),
        pltpu.VMEM((NBG, RPW, DIM), jnp.float32),
        pltpu.VMEM((NBT, DIM // 8, 8, TP), jnp.float32),
        [pltpu.SemaphoreType.DMA] * NBG,
        [pltpu.SemaphoreType.DMA] * NBT,
    ],
    compiler_params=pltpu.CompilerParams(
        use_tc_tiling_on_sc=False, needs_layout_passes=False),
)
def _emb_gather(inpt_hbm, table_hbm, out_hbm, idx_v, gbuf, tbuf, gsems, osems):
    wid = lax.axis_index("s") * NC + lax.axis_index("c")
    b0 = wid * RPW
    pltpu.sync_copy(inpt_hbm.at[:, pl.ds(b0, RPW)], idx_v)

    def gather_start(h, bg):
        pltpu.async_copy(table_hbm.at[idx_v.at[h]], gbuf.at[bg], gsems[bg])

    def gather_wait(h, bg):
        pltpu.make_async_copy(
            table_hbm.at[idx_v.at[h]], gbuf.at[bg], gsems[bg]).wait()

    def out_start(h, bt):
        pltpu.async_copy(tbuf.at[bt, :, :, pl.ds(0, RPW)],
                         out_hbm.at[h, :, wid], osems[bt])

    def out_wait(h, bt):
        pltpu.make_async_copy(
            tbuf.at[bt, :, :, pl.ds(0, RPW)],
            out_hbm.at[h, :, wid], osems[bt]).wait()

    iota = lax.iota(jnp.int32, 16)
    dh_vecs = [(iota + d0) // 8 for d0 in range(0, DIM, 16)]
    dl_vecs = [(iota + d0) % 8 for d0 in range(0, DIM, 16)]
    zeros16 = jnp.full((16,), 0, jnp.int32)

    def transpose_slab(bg, bt):
        @plsc.parallel_loop(0, RPW, unroll=8)
        def _(r):
            r_splat = zeros16 + r
            for k in range(DIM // 16):
                v = gbuf[bg, r, pl.ds(k * 16, 16)]
                plsc.store_scatter(
                    tbuf.at[bt], [dh_vecs[k], dl_vecs[k], r_splat], v)

    def step(h, bg, bt):
        static = isinstance(h, int)
        gather_wait(h, bg)

        if static:
            if h >= NBT:
                out_wait(h - NBT, bt)
        else:
            @pl.when(h >= NBT)
            def _():
                out_wait(h - NBT, bt)

        transpose_slab(bg, bt)
        out_start(h, bt)

        if static:
            if h + NBG < HIST:
                gather_start(h + NBG, bg)
        else:
            @pl.when(h + NBG < HIST)
            def _():
                gather_start(h + NBG, bg)

    for bg in range(NBG):
        gather_start(bg, bg)

    def group(g, carry):
        for b in range(STEP):
            step(g * STEP + b, b % NBG, b % NBT)
        return carry

    lax.fori_loop(0, NGRP, group, 0)

    for h in (48, 49):
        step(h, h % NBG, h % NBT)

    for h in (48, 49):
        out_wait(h, h % NBT)


def kernel(inp, table):
    out5 = _emb_gather(inp.T, table)
    return jnp.transpose(out5, (2, 4, 0, 1, 3)).reshape(BATCH, HIST, DIM)


# speedup vs baseline: 3.1900x; 1.0037x over previous
import functools

import jax
import jax.numpy as jnp
from jax import lax
from jax.experimental import pallas as pl
from jax.experimental.pallas import tpu as pltpu
from jax.experimental.pallas import tpu_sc as plsc

DIM = 64
BATCH = 4096
HIST = 50
NC, NS = 2, 16
NW = NC * NS
RPW = BATCH // NW
NBG = 4
NBT = 2
TP = 129
STEP = 4
NGRP = 48 // STEP

_mesh = plsc.VectorSubcoreMesh(core_axis_name="c", subcore_axis_name="s")


@functools.partial(
    pl.kernel,
    mesh=_mesh,
    out_type=jax.ShapeDtypeStruct((HIST, DIM // 8, NW, 8, RPW), jnp.float32),
    scratch_types=[
        pltpu.VMEM((HIST, RPW), jnp.int32),
        pltpu.VMEM((NBG, RPW, DIM), jnp.float32),
        pltpu.VMEM((NBT, DIM // 8, 8, TP), jnp.float32),
        [pltpu.SemaphoreType.DMA] * NBG,
        [pltpu.SemaphoreType.DMA] * NBT,
    ],
    compiler_params=pltpu.CompilerParams(
        use_tc_tiling_on_sc=False, needs_layout_passes=False),
)
def _emb_gather(inpt_hbm, table_hbm, out_hbm, idx_v, gbuf, tbuf, gsems, osems):
    wid = lax.axis_index("s") * NC + lax.axis_index("c")
    b0 = wid * RPW
    pltpu.sync_copy(inpt_hbm.at[:, pl.ds(b0, RPW)], idx_v)

    def gather_start(h, bg):
        pltpu.async_copy(table_hbm.at[idx_v.at[h]], gbuf.at[bg], gsems[bg])

    def gather_wait(h, bg):
        pltpu.make_async_copy(
            table_hbm.at[idx_v.at[h]], gbuf.at[bg], gsems[bg]).wait()

    def out_start(h, bt):
        pltpu.async_copy(tbuf.at[bt, :, :, pl.ds(0, RPW)],
                         out_hbm.at[h, :, wid], osems[bt])

    def out_wait(h, bt):
        pltpu.make_async_copy(
            tbuf.at[bt, :, :, pl.ds(0, RPW)],
            out_hbm.at[h, :, wid], osems[bt]).wait()

    iota = lax.iota(jnp.int32, 16)
    dh_vecs = [(iota + d0) // 8 for d0 in range(0, DIM, 16)]
    dl_vecs = [(iota + d0) % 8 for d0 in range(0, DIM, 16)]
    zeros16 = jnp.full((16,), 0, jnp.int32)

    def transpose_slab(bg, bt):
        @plsc.parallel_loop(0, RPW, unroll=16)
        def _(r):
            r_splat = zeros16 + r
            for k in range(DIM // 16):
                v = gbuf[bg, r, pl.ds(k * 16, 16)]
                plsc.store_scatter(
                    tbuf.at[bt], [dh_vecs[k], dl_vecs[k], r_splat], v)

    def step(h, bg, bt):
        static = isinstance(h, int)
        gather_wait(h, bg)

        if static:
            if h >= NBT:
                out_wait(h - NBT, bt)
        else:
            @pl.when(h >= NBT)
            def _():
                out_wait(h - NBT, bt)

        transpose_slab(bg, bt)
        out_start(h, bt)

        if static:
            if h + NBG < HIST:
                gather_start(h + NBG, bg)
        else:
            @pl.when(h + NBG < HIST)
            def _():
                gather_start(h + NBG, bg)

    for bg in range(NBG):
        gather_start(bg, bg)

    def group(g, carry):
        for b in range(STEP):
            step(g * STEP + b, b % NBG, b % NBT)
        return carry

    lax.fori_loop(0, NGRP, group, 0)

    for h in (48, 49):
        step(h, h % NBG, h % NBT)

    for h in (48, 49):
        out_wait(h, h % NBT)


def kernel(inp, table):
    out5 = _emb_gather(inp.T, table)
    return jnp.transpose(out5, (2, 4, 0, 1, 3)).reshape(BATCH, HIST, DIM)
